# baseline (speedup 1.0000x reference)
import functools

import jax
import jax.numpy as jnp
from jax.experimental import pallas as pl
from jax.experimental.pallas import tpu as pltpu
from jax.experimental.pallas import tpu_sc as plsc

_P = 1024
_K = 64
_INTERPRET = False


def _fold_mlp(ps):
    folded = []
    s_prev = None
    t_prev = None
    for p in ps:
        W, b = p['W'], p['b']
        if s_prev is not None:
            b = b + t_prev @ W
            W = s_prev[:, None] * W
        folded.append((W, b))
        s = p['g'] / jnp.sqrt(p['rv'] + 1e-5)
        t = p['be'] - p['rm'] * s
        s_prev, t_prev = s, t
    return folded, s_prev, t_prev


def _conv_body(zrows_ref, posq_ref, valid_ref, wq_ref, b1_ref, w2_ref,
               b2_ref, w3_ref, b3_ref, s3_ref, t3_ref, out_ref, *, K):
    qblk = out_ref.shape[0]
    c1 = w2_ref.shape[0]
    c3 = out_ref.shape[1]
    z = zrows_ref[...][:, :c1]
    qb = b1_ref[...] - jnp.dot(posq_ref[...], wq_ref[...],
                               preferred_element_type=jnp.float32)
    h = z.reshape(qblk, K, c1) + qb[:, None, :]
    h = jnp.maximum(h, 0.0).reshape(qblk * K, c1)
    h = jnp.dot(h, w2_ref[...], preferred_element_type=jnp.float32) + b2_ref[...]
    h = jnp.maximum(h, 0.0)
    h = jnp.dot(h, w3_ref[...], preferred_element_type=jnp.float32) + b3_ref[...]
    h = jnp.maximum(h, 0.0)
    h = h * s3_ref[...] + t3_ref[...]
    H = h.reshape(qblk, K, c3)
    H = jnp.where(valid_ref[...][:, :, None] > 0, H, -jnp.inf)
    out_ref[...] = jnp.max(H, axis=1)


def _point_conv(zrows, posq, valid, wq, b1, w2, b2, w3, b3, s3, t3, qblk):
    QK, c1 = zrows.shape
    Q = posq.shape[0]
    K = QK // Q
    c3 = w3.shape[1]
    vec = lambda a: a.reshape(1, -1)
    return pl.pallas_call(
        functools.partial(_conv_body, K=K),
        grid=(Q // qblk,),
        in_specs=[
            pl.BlockSpec((qblk * K, c1), lambda q: (q, 0)),
            pl.BlockSpec((qblk, 3), lambda q: (q, 0)),
            pl.BlockSpec((qblk, K), lambda q: (q, 0)),
            pl.BlockSpec(wq.shape, lambda q: (0, 0)),
            pl.BlockSpec((1, b1.shape[0]), lambda q: (0, 0)),
            pl.BlockSpec(w2.shape, lambda q: (0, 0)),
            pl.BlockSpec((1, b2.shape[0]), lambda q: (0, 0)),
            pl.BlockSpec(w3.shape, lambda q: (0, 0)),
            pl.BlockSpec((1, b3.shape[0]), lambda q: (0, 0)),
            pl.BlockSpec((1, s3.shape[0]), lambda q: (0, 0)),
            pl.BlockSpec((1, t3.shape[0]), lambda q: (0, 0)),
        ],
        out_specs=pl.BlockSpec((qblk, c3), lambda q: (q, 0)),
        out_shape=jax.ShapeDtypeStruct((Q, c3), jnp.float32),
        interpret=_INTERPRET,
    )(zrows, posq, valid, wq, vec(b1), w2, vec(b2), w3, vec(b3),
      vec(s3), vec(t3))


def _tail_body(feat_ref, w1_ref, b1_ref, w2_ref, b2_ref, w3_ref, b3_ref,
               s3_ref, t3_ref, l1w_ref, l1b_ref, l2w_ref, l2b_ref,
               l3w_ref, l3b_ref, out_ref, *, nb, npts):
    h = jnp.dot(feat_ref[...], w1_ref[...], preferred_element_type=jnp.float32) + b1_ref[...]
    h = jnp.maximum(h, 0.0)
    h = jnp.dot(h, w2_ref[...], preferred_element_type=jnp.float32) + b2_ref[...]
    h = jnp.maximum(h, 0.0)
    h = jnp.dot(h, w3_ref[...], preferred_element_type=jnp.float32) + b3_ref[...]
    h = jnp.maximum(h, 0.0)
    h = h * s3_ref[...] + t3_ref[...]
    rows = [jnp.max(h[b * npts:(b + 1) * npts, :], axis=0, keepdims=True)
            for b in range(nb)]
    g = jnp.concatenate(rows, axis=0)
    h = jnp.maximum(jnp.dot(g, l1w_ref[...], preferred_element_type=jnp.float32) + l1b_ref[...], 0.0)
    h = jnp.maximum(jnp.dot(h, l2w_ref[...], preferred_element_type=jnp.float32) + l2b_ref[...], 0.0)
    out_ref[...] = jnp.dot(h, l3w_ref[...], preferred_element_type=jnp.float32) + l3b_ref[...]


def _tail(feat, sa3, lin1, lin2, lin3, nb, npts):
    layers, s3, t3 = _fold_mlp(sa3)
    (w1, b1), (w2, b2), (w3, b3) = layers
    vec = lambda a: a.reshape(1, -1)
    args = (feat, w1, vec(b1), w2, vec(b2), w3, vec(b3), vec(s3), vec(t3),
            lin1['W'], vec(lin1['b']), lin2['W'], vec(lin2['b']),
            lin3['W'], vec(lin3['b']))
    return pl.pallas_call(
        functools.partial(_tail_body, nb=nb, npts=npts),
        out_shape=jax.ShapeDtypeStruct((nb, lin3['W'].shape[1]), jnp.float32),
        interpret=_INTERPRET,
    )(*args)


def _fps_chain(px, py, pz, S):
    B, P = px.shape
    iota = jax.lax.broadcasted_iota(jnp.int32, (B, P), 1)
    iota_s = jax.lax.broadcasted_iota(jnp.int32, (B, S), 1)
    sx, sy, sz = px[:, 0:1], py[:, 0:1], pz[:, 0:1]
    zq = jnp.zeros((B, S), jnp.float32)
    first = iota_s == 0
    qx = jnp.where(first, sx, zq)
    qy = jnp.where(first, sy, zq)
    qz = jnp.where(first, sz, zq)
    d0 = (px - sx) ** 2 + (py - sy) ** 2 + (pz - sz) ** 2

    def body(i, carry):
        dists, qx, qy, qz = carry
        m = jnp.max(dists, axis=1, keepdims=True)
        eq = dists == m
        idx = jnp.min(jnp.where(eq, iota, P), axis=1, keepdims=True)
        onehot = iota == idx
        sx = jnp.sum(jnp.where(onehot, px, 0.0), axis=1, keepdims=True)
        sy = jnp.sum(jnp.where(onehot, py, 0.0), axis=1, keepdims=True)
        sz = jnp.sum(jnp.where(onehot, pz, 0.0), axis=1, keepdims=True)
        slot = iota_s == i
        qx = jnp.where(slot, sx, qx)
        qy = jnp.where(slot, sy, qy)
        qz = jnp.where(slot, sz, qz)
        d_new = (px - sx) ** 2 + (py - sy) ** 2 + (pz - sz) ** 2
        return (jnp.minimum(dists, d_new), qx, qy, qz)

    _, qx, qy, qz = jax.lax.fori_loop(1, S, body, (d0, qx, qy, qz))
    return qx, qy, qz


def _fps_body(px_ref, py_ref, pz_ref,
              q1x_ref, q1y_ref, q1z_ref, q2x_ref, q2y_ref, q2z_ref,
              *, S1, S2):
    q1x, q1y, q1z = _fps_chain(px_ref[...], py_ref[...], pz_ref[...], S1)
    q1x_ref[...] = q1x
    q1y_ref[...] = q1y
    q1z_ref[...] = q1z
    q2x, q2y, q2z = _fps_chain(q1x, q1y, q1z, S2)
    q2x_ref[...] = q2x
    q2y_ref[...] = q2y
    q2z_ref[...] = q2z


def _fps_both(pos_b, S1, S2):
    B = pos_b.shape[0]
    px = pos_b[:, :, 0]
    py = pos_b[:, :, 1]
    pz = pos_b[:, :, 2]
    outs = pl.pallas_call(
        functools.partial(_fps_body, S1=S1, S2=S2),
        out_shape=[jax.ShapeDtypeStruct((B, S1), jnp.float32)] * 3
        + [jax.ShapeDtypeStruct((B, S2), jnp.float32)] * 3,
        interpret=_INTERPRET,
    )(px, py, pz)
    q1 = jnp.stack(outs[:3], axis=-1)
    q2 = jnp.stack(outs[3:], axis=-1)
    return q1, q2


def _sc_gather(tab, idx, nbuf):
    V, C = tab.shape
    R = idx.shape[0]
    info = plsc.get_sparse_core_info()
    NW = info.num_cores * info.num_subcores
    CH = R // (NW * 128)
    assert CH % nbuf == 0
    idx2d = idx.reshape(NW * CH, 128)
    mesh = plsc.VectorSubcoreMesh(core_axis_name="c", subcore_axis_name="s")

    @functools.partial(
        pl.kernel, mesh=mesh,
        out_type=jax.ShapeDtypeStruct((R, C), jnp.float32),
        scratch_types=[
            pltpu.VMEM((CH, 128), jnp.int32),
            pltpu.VMEM((nbuf, 128, C), jnp.float32),
            pltpu.SemaphoreType.DMA,
            pltpu.SemaphoreType.DMA,
        ],
    )
    def gk(tab_hbm, idx_hbm, out_hbm, idx_v, rbuf, gsem, ssem):
        wid = jax.lax.axis_index("s") * info.num_cores + jax.lax.axis_index("c")
        base = wid * CH
        pltpu.sync_copy(idx_hbm.at[pl.ds(base, CH)], idx_v)
        scat = []
        for sc in range(CH // nbuf):
            for h in scat:
                h.wait()
            scat = []
            gath = []
            for b in range(nbuf):
                j = sc * nbuf + b
                gath.append(pltpu.async_copy(
                    tab_hbm.at[idx_v.at[j]], rbuf.at[b], gsem))
            for b in range(nbuf):
                j = sc * nbuf + b
                gath[b].wait()
                scat.append(pltpu.async_copy(
                    rbuf.at[b], out_hbm.at[pl.ds((base + j) * 128, 128)], ssem))
        for h in scat:
            h.wait()

    return gk(tab, idx2d)


def _neighbors(pos_q, pos_b, r):
    d2 = jnp.sum((pos_q[:, :, None, :] - pos_b[:, None, :, :]) ** 2, axis=-1)
    neg = jnp.where(d2 <= r * r, -d2, -jnp.inf)
    vals, nbr = jax.lax.top_k(neg, _K)
    valid = vals > -jnp.inf
    return nbr, valid


def _sa_stage(ps, x_b, pos_b, pos_q, r, qblk):
    Bc, Pc, _ = pos_b.shape
    S = pos_q.shape[1]
    nbr, valid = _neighbors(pos_q, pos_b, r)
    layers, s3, t3 = _fold_mlp(ps)
    (w1, b1), (w2, b2), (w3, b3) = layers
    tab = jnp.concatenate(
        [x_b.reshape(Bc * Pc, -1), pos_b.reshape(Bc * Pc, 3)], axis=1) @ w1
    if tab.shape[1] % 128:
        tab = jnp.concatenate(
            [tab, jnp.zeros((tab.shape[0], 128 - tab.shape[1] % 128),
                            jnp.float32)], axis=1)
    gidx = (nbr + (jnp.arange(Bc, dtype=jnp.int32) * Pc)[:, None, None])
    zrows = _sc_gather(tab, gidx.reshape(-1), nbuf=4)
    out = _point_conv(zrows, pos_q.reshape(Bc * S, 3),
                      valid.reshape(Bc * S, _K).astype(jnp.float32),
                      w1[-3:], b1, w2, b2, w3, b3, s3, t3, qblk)
    return out.reshape(Bc, S, -1)


def kernel(x, pos, batch, params):
    Bn = batch.shape[0] // _P
    Pn = x.shape[0] // Bn
    x_b = x.reshape(Bn, Pn, -1)
    pos_b = pos.reshape(Bn, Pn, 3)
    pos_q1, pos_q2 = _fps_both(pos_b, Pn // 2, Pn // 8)
    d2a = jnp.sum((pos_q1[:, :, None, :] - pos_b[:, None, :, :]) ** 2, axis=-1)
    nega = jnp.where(d2a <= 0.04, -d2a, -jnp.inf)
    G = 4
    negg = nega.reshape(Bn, 512, G, 1024 // G)
    gv, gi = jax.lax.top_k(negg, _K)
    cand = gv.reshape(Bn, 512, G * _K)
    v2, i2 = jax.lax.top_k(cand, _K)
    gidx = (gi + (jnp.arange(G, dtype=jnp.int32) * (1024 // G))[None, None, :, None]
            ).reshape(Bn, 512, G * _K)
    nbr1 = jnp.take_along_axis(gidx, i2, axis=-1)
    return (jnp.zeros((Bn, 40), jnp.float32) + jnp.sum(v2) + jnp.sum(nbr1))
    x1 = _sa_stage(params['sa1'], x_b, pos_b, pos_q1, 0.2, qblk=128)
    x2 = _sa_stage(params['sa2'], x1, pos_q1, pos_q2, 0.4, qblk=128)
    feat = jnp.concatenate([x2, pos_q2], axis=-1)
    nb, npts, c = feat.shape
    return _tail(feat.reshape(nb * npts, c), params['sa3'],
                 params['lin1'], params['lin2'], params['lin3'], nb, npts)


# speedup vs baseline: 9.5709x; 9.5709x over previous
import functools

import jax
import jax.numpy as jnp
from jax.experimental import pallas as pl
from jax.experimental.pallas import tpu as pltpu
from jax.experimental.pallas import tpu_sc as plsc

_P = 1024
_K = 64
_INTERPRET = False


def _fold_mlp(ps):
    folded = []
    s_prev = None
    t_prev = None
    for p in ps:
        W, b = p['W'], p['b']
        if s_prev is not None:
            b = b + t_prev @ W
            W = s_prev[:, None] * W
        folded.append((W, b))
        s = p['g'] / jnp.sqrt(p['rv'] + 1e-5)
        t = p['be'] - p['rm'] * s
        s_prev, t_prev = s, t
    return folded, s_prev, t_prev


def _conv_body(zrows_ref, posq_ref, valid_ref, wq_ref, b1_ref, w2_ref,
               b2_ref, w3_ref, b3_ref, s3_ref, t3_ref, out_ref, *, K):
    qblk = out_ref.shape[0]
    c1 = w2_ref.shape[0]
    c3 = out_ref.shape[1]
    z = zrows_ref[...][:, :c1]
    qb = b1_ref[...] - jnp.dot(posq_ref[...], wq_ref[...],
                               preferred_element_type=jnp.float32)
    h = z.reshape(qblk, K, c1) + qb[:, None, :]
    h = jnp.maximum(h, 0.0).reshape(qblk * K, c1)
    h = jnp.dot(h, w2_ref[...], preferred_element_type=jnp.float32) + b2_ref[...]
    h = jnp.maximum(h, 0.0)
    h = jnp.dot(h, w3_ref[...], preferred_element_type=jnp.float32) + b3_ref[...]
    h = jnp.maximum(h, 0.0)
    h = h * s3_ref[...] + t3_ref[...]
    H = h.reshape(qblk, K, c3)
    H = jnp.where(valid_ref[...][:, :, None] > 0, H, -jnp.inf)
    out_ref[...] = jnp.max(H, axis=1)


def _point_conv(zrows, posq, valid, wq, b1, w2, b2, w3, b3, s3, t3, qblk):
    QK, c1 = zrows.shape
    Q = posq.shape[0]
    K = QK // Q
    c3 = w3.shape[1]
    vec = lambda a: a.reshape(1, -1)
    return pl.pallas_call(
        functools.partial(_conv_body, K=K),
        grid=(Q // qblk,),
        in_specs=[
            pl.BlockSpec((qblk * K, c1), lambda q: (q, 0)),
            pl.BlockSpec((qblk, 3), lambda q: (q, 0)),
            pl.BlockSpec((qblk, K), lambda q: (q, 0)),
            pl.BlockSpec(wq.shape, lambda q: (0, 0)),
            pl.BlockSpec((1, b1.shape[0]), lambda q: (0, 0)),
            pl.BlockSpec(w2.shape, lambda q: (0, 0)),
            pl.BlockSpec((1, b2.shape[0]), lambda q: (0, 0)),
            pl.BlockSpec(w3.shape, lambda q: (0, 0)),
            pl.BlockSpec((1, b3.shape[0]), lambda q: (0, 0)),
            pl.BlockSpec((1, s3.shape[0]), lambda q: (0, 0)),
            pl.BlockSpec((1, t3.shape[0]), lambda q: (0, 0)),
        ],
        out_specs=pl.BlockSpec((qblk, c3), lambda q: (q, 0)),
        out_shape=jax.ShapeDtypeStruct((Q, c3), jnp.float32),
        interpret=_INTERPRET,
    )(zrows, posq, valid, wq, vec(b1), w2, vec(b2), w3, vec(b3),
      vec(s3), vec(t3))


def _tail_body(feat_ref, w1_ref, b1_ref, w2_ref, b2_ref, w3_ref, b3_ref,
               s3_ref, t3_ref, l1w_ref, l1b_ref, l2w_ref, l2b_ref,
               l3w_ref, l3b_ref, out_ref, *, nb, npts):
    h = jnp.dot(feat_ref[...], w1_ref[...], preferred_element_type=jnp.float32) + b1_ref[...]
    h = jnp.maximum(h, 0.0)
    h = jnp.dot(h, w2_ref[...], preferred_element_type=jnp.float32) + b2_ref[...]
    h = jnp.maximum(h, 0.0)
    h = jnp.dot(h, w3_ref[...], preferred_element_type=jnp.float32) + b3_ref[...]
    h = jnp.maximum(h, 0.0)
    h = h * s3_ref[...] + t3_ref[...]
    rows = [jnp.max(h[b * npts:(b + 1) * npts, :], axis=0, keepdims=True)
            for b in range(nb)]
    g = jnp.concatenate(rows, axis=0)
    h = jnp.maximum(jnp.dot(g, l1w_ref[...], preferred_element_type=jnp.float32) + l1b_ref[...], 0.0)
    h = jnp.maximum(jnp.dot(h, l2w_ref[...], preferred_element_type=jnp.float32) + l2b_ref[...], 0.0)
    out_ref[...] = jnp.dot(h, l3w_ref[...], preferred_element_type=jnp.float32) + l3b_ref[...]


def _tail(feat, sa3, lin1, lin2, lin3, nb, npts):
    layers, s3, t3 = _fold_mlp(sa3)
    (w1, b1), (w2, b2), (w3, b3) = layers
    vec = lambda a: a.reshape(1, -1)
    args = (feat, w1, vec(b1), w2, vec(b2), w3, vec(b3), vec(s3), vec(t3),
            lin1['W'], vec(lin1['b']), lin2['W'], vec(lin2['b']),
            lin3['W'], vec(lin3['b']))
    return pl.pallas_call(
        functools.partial(_tail_body, nb=nb, npts=npts),
        out_shape=jax.ShapeDtypeStruct((nb, lin3['W'].shape[1]), jnp.float32),
        interpret=_INTERPRET,
    )(*args)


def _fps_chain(px, py, pz, S):
    B, P = px.shape
    iota = jax.lax.broadcasted_iota(jnp.int32, (B, P), 1)
    iota_s = jax.lax.broadcasted_iota(jnp.int32, (B, S), 1)
    sx, sy, sz = px[:, 0:1], py[:, 0:1], pz[:, 0:1]
    zq = jnp.zeros((B, S), jnp.float32)
    first = iota_s == 0
    qx = jnp.where(first, sx, zq)
    qy = jnp.where(first, sy, zq)
    qz = jnp.where(first, sz, zq)
    d0 = (px - sx) ** 2 + (py - sy) ** 2 + (pz - sz) ** 2

    def body(i, carry):
        dists, qx, qy, qz = carry
        m = jnp.max(dists, axis=1, keepdims=True)
        eq = dists == m
        idx = jnp.min(jnp.where(eq, iota, P), axis=1, keepdims=True)
        onehot = iota == idx
        sx = jnp.sum(jnp.where(onehot, px, 0.0), axis=1, keepdims=True)
        sy = jnp.sum(jnp.where(onehot, py, 0.0), axis=1, keepdims=True)
        sz = jnp.sum(jnp.where(onehot, pz, 0.0), axis=1, keepdims=True)
        slot = iota_s == i
        qx = jnp.where(slot, sx, qx)
        qy = jnp.where(slot, sy, qy)
        qz = jnp.where(slot, sz, qz)
        d_new = (px - sx) ** 2 + (py - sy) ** 2 + (pz - sz) ** 2
        return (jnp.minimum(dists, d_new), qx, qy, qz)

    _, qx, qy, qz = jax.lax.fori_loop(1, S, body, (d0, qx, qy, qz))
    return qx, qy, qz


def _fps_body(px_ref, py_ref, pz_ref,
              q1x_ref, q1y_ref, q1z_ref, q2x_ref, q2y_ref, q2z_ref,
              *, S1, S2):
    q1x, q1y, q1z = _fps_chain(px_ref[...], py_ref[...], pz_ref[...], S1)
    q1x_ref[...] = q1x
    q1y_ref[...] = q1y
    q1z_ref[...] = q1z
    q2x, q2y, q2z = _fps_chain(q1x, q1y, q1z, S2)
    q2x_ref[...] = q2x
    q2y_ref[...] = q2y
    q2z_ref[...] = q2z


def _fps_both(pos_b, S1, S2):
    B = pos_b.shape[0]
    px = pos_b[:, :, 0]
    py = pos_b[:, :, 1]
    pz = pos_b[:, :, 2]
    outs = pl.pallas_call(
        functools.partial(_fps_body, S1=S1, S2=S2),
        out_shape=[jax.ShapeDtypeStruct((B, S1), jnp.float32)] * 3
        + [jax.ShapeDtypeStruct((B, S2), jnp.float32)] * 3,
        interpret=_INTERPRET,
    )(px, py, pz)
    q1 = jnp.stack(outs[:3], axis=-1)
    q2 = jnp.stack(outs[3:], axis=-1)
    return q1, q2


def _sc_gather(tab, idx, nbuf):
    V, C = tab.shape
    R = idx.shape[0]
    info = plsc.get_sparse_core_info()
    NW = info.num_cores * info.num_subcores
    CH = R // (NW * 128)
    assert CH % nbuf == 0
    idx2d = idx.reshape(NW * CH, 128)
    mesh = plsc.VectorSubcoreMesh(core_axis_name="c", subcore_axis_name="s")

    @functools.partial(
        pl.kernel, mesh=mesh,
        out_type=jax.ShapeDtypeStruct((R, C), jnp.float32),
        scratch_types=[
            pltpu.VMEM((CH, 128), jnp.int32),
            pltpu.VMEM((nbuf, 128, C), jnp.float32),
            pltpu.SemaphoreType.DMA,
            pltpu.SemaphoreType.DMA,
        ],
    )
    def gk(tab_hbm, idx_hbm, out_hbm, idx_v, rbuf, gsem, ssem):
        wid = jax.lax.axis_index("s") * info.num_cores + jax.lax.axis_index("c")
        base = wid * CH
        pltpu.sync_copy(idx_hbm.at[pl.ds(base, CH)], idx_v)
        scat = []
        for sc in range(CH // nbuf):
            for h in scat:
                h.wait()
            scat = []
            gath = []
            for b in range(nbuf):
                j = sc * nbuf + b
                gath.append(pltpu.async_copy(
                    tab_hbm.at[idx_v.at[j]], rbuf.at[b], gsem))
            for b in range(nbuf):
                j = sc * nbuf + b
                gath[b].wait()
                scat.append(pltpu.async_copy(
                    rbuf.at[b], out_hbm.at[pl.ds((base + j) * 128, 128)], ssem))
        for h in scat:
            h.wait()

    return gk(tab, idx2d)


def _neighbors(pos_q, pos_b, r):
    d2 = jnp.sum((pos_q[:, :, None, :] - pos_b[:, None, :, :]) ** 2, axis=-1)
    neg = jnp.where(d2 <= r * r, -d2, -jnp.inf)
    vals, nbr = jax.lax.top_k(neg, _K)
    valid = vals > -jnp.inf
    return nbr, valid


def _sa_stage(ps, x_b, pos_b, pos_q, r, qblk):
    Bc, Pc, _ = pos_b.shape
    S = pos_q.shape[1]
    nbr, valid = _neighbors(pos_q, pos_b, r)
    layers, s3, t3 = _fold_mlp(ps)
    (w1, b1), (w2, b2), (w3, b3) = layers
    tab = jnp.concatenate(
        [x_b.reshape(Bc * Pc, -1), pos_b.reshape(Bc * Pc, 3)], axis=1) @ w1
    if tab.shape[1] % 128:
        tab = jnp.concatenate(
            [tab, jnp.zeros((tab.shape[0], 128 - tab.shape[1] % 128),
                            jnp.float32)], axis=1)
    gidx = (nbr + (jnp.arange(Bc, dtype=jnp.int32) * Pc)[:, None, None])
    zrows = _sc_gather(tab, gidx.reshape(-1), nbuf=4)
    out = _point_conv(zrows, pos_q.reshape(Bc * S, 3),
                      valid.reshape(Bc * S, _K).astype(jnp.float32),
                      w1[-3:], b1, w2, b2, w3, b3, s3, t3, qblk)
    return out.reshape(Bc, S, -1)


def kernel(x, pos, batch, params):
    Bn = batch.shape[0] // _P
    Pn = x.shape[0] // Bn
    x_b = x.reshape(Bn, Pn, -1)
    pos_b = pos.reshape(Bn, Pn, 3)
    pos_q1, pos_q2 = _fps_both(pos_b, Pn // 2, Pn // 8)
    x1 = _sa_stage(params['sa1'], x_b, pos_b, pos_q1, 0.2, qblk=128)
    x2 = _sa_stage(params['sa2'], x1, pos_q1, pos_q2, 0.4, qblk=128)
    feat = jnp.concatenate([x2, pos_q2], axis=-1)
    nb, npts, c = feat.shape
    return _tail(feat.reshape(nb * npts, c), params['sa3'],
                 params['lin1'], params['lin2'], params['lin3'], nb, npts)


# speedup vs baseline: 13.0235x; 1.3607x over previous
import functools

import jax
import jax.numpy as jnp
from jax.experimental import pallas as pl
from jax.experimental.pallas import tpu as pltpu
from jax.experimental.pallas import tpu_sc as plsc

_P = 1024
_K = 64


def _fold_mlp(ps):
    folded = []
    s_prev = None
    t_prev = None
    for p in ps:
        W, b = p['W'], p['b']
        if s_prev is not None:
            b = b + t_prev @ W
            W = s_prev[:, None] * W
        folded.append((W, b))
        s = p['g'] / jnp.sqrt(p['rv'] + 1e-5)
        t = p['be'] - p['rm'] * s
        s_prev, t_prev = s, t
    return folded, s_prev, t_prev


def _conv_body(zrows_ref, posq_ref, valid_ref, wq_ref, b1_ref, w2_ref,
               b2_ref, w3_ref, b3_ref, s3_ref, t3_ref, out_ref, *, K):
    qblk = out_ref.shape[0]
    c1 = w2_ref.shape[0]
    c3 = out_ref.shape[1]
    z = zrows_ref[...][:, :c1]
    qb = b1_ref[...] - jnp.dot(posq_ref[...], wq_ref[...],
                               preferred_element_type=jnp.float32)
    h = z.reshape(qblk, K, c1) + qb[:, None, :]
    h = jnp.maximum(h, 0.0).reshape(qblk * K, c1)
    h = jnp.dot(h, w2_ref[...], preferred_element_type=jnp.float32) + b2_ref[...]
    h = jnp.maximum(h, 0.0)
    h = jnp.dot(h, w3_ref[...], preferred_element_type=jnp.float32) + b3_ref[...]
    h = jnp.maximum(h, 0.0)
    h = h * s3_ref[...] + t3_ref[...]
    H = h.reshape(qblk, K, c3)
    H = jnp.where(valid_ref[...][:, :, None] > 0, H, -jnp.inf)
    out_ref[...] = jnp.max(H, axis=1)


def _point_conv(zrows, posq, valid, wq, b1, w2, b2, w3, b3, s3, t3, qblk):
    QK, c1 = zrows.shape
    Q = posq.shape[0]
    K = QK // Q
    c3 = w3.shape[1]
    vec = lambda a: a.reshape(1, -1)
    return pl.pallas_call(
        functools.partial(_conv_body, K=K),
        grid=(Q // qblk,),
        in_specs=[
            pl.BlockSpec((qblk * K, c1), lambda q: (q, 0)),
            pl.BlockSpec((qblk, 3), lambda q: (q, 0)),
            pl.BlockSpec((qblk, K), lambda q: (q, 0)),
            pl.BlockSpec(wq.shape, lambda q: (0, 0)),
            pl.BlockSpec((1, b1.shape[0]), lambda q: (0, 0)),
            pl.BlockSpec(w2.shape, lambda q: (0, 0)),
            pl.BlockSpec((1, b2.shape[0]), lambda q: (0, 0)),
            pl.BlockSpec(w3.shape, lambda q: (0, 0)),
            pl.BlockSpec((1, b3.shape[0]), lambda q: (0, 0)),
            pl.BlockSpec((1, s3.shape[0]), lambda q: (0, 0)),
            pl.BlockSpec((1, t3.shape[0]), lambda q: (0, 0)),
        ],
        out_specs=pl.BlockSpec((qblk, c3), lambda q: (q, 0)),
        out_shape=jax.ShapeDtypeStruct((Q, c3), jnp.float32),
        interpret=False,
    )(zrows, posq, valid, wq, vec(b1), w2, vec(b2), w3, vec(b3),
      vec(s3), vec(t3))


def _tail_body(feat_ref, w1_ref, b1_ref, w2_ref, b2_ref, w3_ref, b3_ref,
               s3_ref, t3_ref, l1w_ref, l1b_ref, l2w_ref, l2b_ref,
               l3w_ref, l3b_ref, out_ref, *, nb, npts):
    h = jnp.dot(feat_ref[...], w1_ref[...], preferred_element_type=jnp.float32) + b1_ref[...]
    h = jnp.maximum(h, 0.0)
    h = jnp.dot(h, w2_ref[...], preferred_element_type=jnp.float32) + b2_ref[...]
    h = jnp.maximum(h, 0.0)
    h = jnp.dot(h, w3_ref[...], preferred_element_type=jnp.float32) + b3_ref[...]
    h = jnp.maximum(h, 0.0)
    h = h * s3_ref[...] + t3_ref[...]
    rows = [jnp.max(h[b * npts:(b + 1) * npts, :], axis=0, keepdims=True)
            for b in range(nb)]
    g = jnp.concatenate(rows, axis=0)
    h = jnp.maximum(jnp.dot(g, l1w_ref[...], preferred_element_type=jnp.float32) + l1b_ref[...], 0.0)
    h = jnp.maximum(jnp.dot(h, l2w_ref[...], preferred_element_type=jnp.float32) + l2b_ref[...], 0.0)
    out_ref[...] = jnp.dot(h, l3w_ref[...], preferred_element_type=jnp.float32) + l3b_ref[...]


def _tail(feat, sa3, lin1, lin2, lin3, nb, npts):
    layers, s3, t3 = _fold_mlp(sa3)
    (w1, b1), (w2, b2), (w3, b3) = layers
    vec = lambda a: a.reshape(1, -1)
    args = (feat, w1, vec(b1), w2, vec(b2), w3, vec(b3), vec(s3), vec(t3),
            lin1['W'], vec(lin1['b']), lin2['W'], vec(lin2['b']),
            lin3['W'], vec(lin3['b']))
    return pl.pallas_call(
        functools.partial(_tail_body, nb=nb, npts=npts),
        out_shape=jax.ShapeDtypeStruct((nb, lin3['W'].shape[1]), jnp.float32),
        interpret=False,
    )(*args)


def _fps_chain(px, py, pz, S):
    B, P = px.shape
    iota = jax.lax.broadcasted_iota(jnp.int32, (B, P), 1)
    iota_s = jax.lax.broadcasted_iota(jnp.int32, (B, S), 1)
    sx, sy, sz = px[:, 0:1], py[:, 0:1], pz[:, 0:1]
    zq = jnp.zeros((B, S), jnp.float32)
    first = iota_s == 0
    qx = jnp.where(first, sx, zq)
    qy = jnp.where(first, sy, zq)
    qz = jnp.where(first, sz, zq)
    d0 = (px - sx) ** 2 + (py - sy) ** 2 + (pz - sz) ** 2

    def body(i, carry):
        dists, qx, qy, qz = carry
        m = jnp.max(dists, axis=1, keepdims=True)
        eq = dists == m
        idx = jnp.min(jnp.where(eq, iota, P), axis=1, keepdims=True)
        onehot = iota == idx
        sx = jnp.sum(jnp.where(onehot, px, 0.0), axis=1, keepdims=True)
        sy = jnp.sum(jnp.where(onehot, py, 0.0), axis=1, keepdims=True)
        sz = jnp.sum(jnp.where(onehot, pz, 0.0), axis=1, keepdims=True)
        slot = iota_s == i
        qx = jnp.where(slot, sx, qx)
        qy = jnp.where(slot, sy, qy)
        qz = jnp.where(slot, sz, qz)
        d_new = (px - sx) ** 2 + (py - sy) ** 2 + (pz - sz) ** 2
        return (jnp.minimum(dists, d_new), qx, qy, qz)

    _, qx, qy, qz = jax.lax.fori_loop(1, S, body, (d0, qx, qy, qz))
    return qx, qy, qz


def _fps_body(px_ref, py_ref, pz_ref,
              q1x_ref, q1y_ref, q1z_ref, q2x_ref, q2y_ref, q2z_ref,
              *, S1, S2):
    q1x, q1y, q1z = _fps_chain(px_ref[...], py_ref[...], pz_ref[...], S1)
    q1x_ref[...] = q1x
    q1y_ref[...] = q1y
    q1z_ref[...] = q1z
    q2x, q2y, q2z = _fps_chain(q1x, q1y, q1z, S2)
    q2x_ref[...] = q2x
    q2y_ref[...] = q2y
    q2z_ref[...] = q2z


def _fps_both(pos_b, S1, S2):
    B = pos_b.shape[0]
    px = pos_b[:, :, 0]
    py = pos_b[:, :, 1]
    pz = pos_b[:, :, 2]
    outs = pl.pallas_call(
        functools.partial(_fps_body, S1=S1, S2=S2),
        out_shape=[jax.ShapeDtypeStruct((B, S1), jnp.float32)] * 3
        + [jax.ShapeDtypeStruct((B, S2), jnp.float32)] * 3,
        interpret=False,
    )(px, py, pz)
    q1 = jnp.stack(outs[:3], axis=-1)
    q2 = jnp.stack(outs[3:], axis=-1)
    return q1, q2


def _sc_gather(tab, idx, nbuf):
    V, C = tab.shape
    R = idx.shape[0]
    info = plsc.get_sparse_core_info()
    NW = info.num_cores * info.num_subcores
    CH = R // (NW * 128)
    assert CH % nbuf == 0
    idx2d = idx.reshape(NW * CH, 128)
    mesh = plsc.VectorSubcoreMesh(core_axis_name="c", subcore_axis_name="s")

    @functools.partial(
        pl.kernel, mesh=mesh,
        out_type=jax.ShapeDtypeStruct((R, C), jnp.float32),
        scratch_types=[
            pltpu.VMEM((CH, 128), jnp.int32),
            pltpu.VMEM((nbuf, 128, C), jnp.float32),
            pltpu.SemaphoreType.DMA,
            pltpu.SemaphoreType.DMA,
        ],
    )
    def gk(tab_hbm, idx_hbm, out_hbm, idx_v, rbuf, gsem, ssem):
        wid = jax.lax.axis_index("s") * info.num_cores + jax.lax.axis_index("c")
        base = wid * CH
        pltpu.sync_copy(idx_hbm.at[pl.ds(base, CH)], idx_v)
        scat = []
        for sc in range(CH // nbuf):
            for h in scat:
                h.wait()
            scat = []
            gath = []
            for b in range(nbuf):
                j = sc * nbuf + b
                gath.append(pltpu.async_copy(
                    tab_hbm.at[idx_v.at[j]], rbuf.at[b], gsem))
            for b in range(nbuf):
                j = sc * nbuf + b
                gath[b].wait()
                scat.append(pltpu.async_copy(
                    rbuf.at[b], out_hbm.at[pl.ds((base + j) * 128, 128)], ssem))
        for h in scat:
            h.wait()

    return gk(tab, idx2d)


def _neighbors(pos_q, pos_b, r):
    d2 = jnp.sum((pos_q[:, :, None, :] - pos_b[:, None, :, :]) ** 2, axis=-1)
    neg = jnp.where(d2 <= r * r, -d2, -jnp.inf)
    vals, nbr = jax.lax.approx_max_k(neg, _K, recall_target=1.0,
                                     reduction_input_size_override=-1,
                                     aggregate_to_topk=True)
    valid = vals > -jnp.inf
    return nbr, valid


def _sa_stage(ps, x_b, pos_b, pos_q, r, qblk):
    Bc, Pc, _ = pos_b.shape
    S = pos_q.shape[1]
    nbr, valid = _neighbors(pos_q, pos_b, r)
    layers, s3, t3 = _fold_mlp(ps)
    (w1, b1), (w2, b2), (w3, b3) = layers
    tab = jnp.concatenate(
        [x_b.reshape(Bc * Pc, -1), pos_b.reshape(Bc * Pc, 3)], axis=1) @ w1
    if tab.shape[1] % 128:
        tab = jnp.concatenate(
            [tab, jnp.zeros((tab.shape[0], 128 - tab.shape[1] % 128),
                            jnp.float32)], axis=1)
    gidx = (nbr + (jnp.arange(Bc, dtype=jnp.int32) * Pc)[:, None, None])
    zrows = _sc_gather(tab, gidx.reshape(-1), nbuf=4)
    out = _point_conv(zrows, pos_q.reshape(Bc * S, 3),
                      valid.reshape(Bc * S, _K).astype(jnp.float32),
                      w1[-3:], b1, w2, b2, w3, b3, s3, t3, qblk)
    return out.reshape(Bc, S, -1)


def kernel(x, pos, batch, params):
    Bn = batch.shape[0] // _P
    Pn = x.shape[0] // Bn
    x_b = x.reshape(Bn, Pn, -1)
    pos_b = pos.reshape(Bn, Pn, 3)
    pos_q1, pos_q2 = _fps_both(pos_b, Pn // 2, Pn // 8)
    x1 = _sa_stage(params['sa1'], x_b, pos_b, pos_q1, 0.2, qblk=128)
    x2 = _sa_stage(params['sa2'], x1, pos_q1, pos_q2, 0.4, qblk=128)
    feat = jnp.concatenate([x2, pos_q2], axis=-1)
    nb, npts, c = feat.shape
    return _tail(feat.reshape(nb * npts, c), params['sa3'],
                 params['lin1'], params['lin2'], params['lin3'], nb, npts)
